# baseline (speedup 1.0000x reference)
import functools

import jax
import jax.numpy as jnp
from jax import lax
from jax.experimental import pallas as pl
from jax.experimental.pallas import tpu as pltpu
from jax.experimental.pallas import tpu_sc as plsc

NC, NS, LANES = 2, 16, 16
NW = NC * NS

INPUT_BITS = 4096
N_IN = 1024
N_STATE = 512
N_OUT = 512
NBITS = N_IN + N_STATE
K = 14
KP = 16
MEM = 1 << K
B = 1024

_MESH = plsc.VectorSubcoreMesh(
    core_axis_name="c", subcore_axis_name="s", num_cores=NC, num_subcores=NS
)


def _make_layer(mode, rows_out, src_rows):
    rows_per_sub = rows_out // NW
    out_dtype = jnp.float32 if mode == "out" else jnp.int32

    def body(src, conn, mem, out, idx16, rows16, addr2d, vals1d, outrow,
             tmprow, sem_rows, sem_gat):
        wid = lax.axis_index("s") * NC + lax.axis_index("c")

        def compute(row, neuron):
            pltpu.sync_copy(conn.at[neuron], idx16)
            pltpu.async_copy(src.at[idx16], rows16, sem_rows).wait()
            for c in range(B // LANES):
                a = rows16[0, pl.ds(c * LANES, LANES)]
                for k in range(1, K):
                    a = a + (rows16[k, pl.ds(c * LANES, LANES)] << k)
                a = a + neuron * MEM
                addr2d[c // 8, pl.ds((c % 8) * LANES, LANES)] = a
            cps = [
                pltpu.async_copy(mem.at[addr2d.at[r]],
                                 vals1d.at[pl.ds(r * 128, 128)], sem_gat)
                for r in range(8)
            ]
            for cp in cps:
                cp.wait()
            if mode == "out":
                pltpu.sync_copy(vals1d, out.at[row])
            else:
                for c in range(B // LANES):
                    v = vals1d[pl.ds(c * LANES, LANES)]
                    outrow[pl.ds(c * LANES, LANES)] = jnp.where(
                        v > 0.5, jnp.int32(1), jnp.int32(0))
                pltpu.sync_copy(outrow, out.at[row])

        if mode == "input":
            for c in range(B // LANES):
                tmprow[pl.ds(c * LANES, LANES)] = jnp.zeros((LANES,), jnp.int32)

        @pl.loop(0, rows_per_sub)
        def _(j):
            row = j * NW + wid
            if mode == "out":
                compute(row, row)
            elif mode == "input":
                @pl.when(row < N_IN)
                def _():
                    compute(row, row)

                @pl.when(row >= N_IN)
                def _():
                    pltpu.sync_copy(tmprow, out.at[row])
            else:
                @pl.when(row < N_IN)
                def _():
                    pltpu.sync_copy(src.at[row], tmprow)
                    pltpu.sync_copy(tmprow, out.at[row])

                @pl.when(row >= N_IN)
                def _():
                    compute(row, row - N_IN)

    return pl.kernel(
        body,
        out_type=jax.ShapeDtypeStruct((rows_out, B), out_dtype),
        mesh=_MESH,
        scratch_types=[
            pltpu.VMEM((KP,), jnp.int32),
            pltpu.VMEM((KP, B), jnp.int32),
            pltpu.VMEM((8, 128), jnp.int32),
            pltpu.VMEM((B,), jnp.float32),
            pltpu.VMEM((B,), jnp.int32),
            pltpu.VMEM((B,), jnp.int32),
            pltpu.SemaphoreType.DMA,
            pltpu.SemaphoreType.DMA,
        ],
        name=f"ram_layer_{mode}",
    )


def _pad_conn(conn):
    n, k = conn.shape
    return jnp.concatenate(
        [conn.astype(jnp.int32), jnp.zeros((n, KP - k), jnp.int32)], axis=1)


@jax.jit
def kernel(x, conn_in, conn_state, conn_out, mem_in, mem_state, mem_out):
    xT = x.astype(jnp.int32).T
    conn_in_p = _pad_conn(conn_in)
    conn_state_p = _pad_conn(conn_state)
    conn_out_p = _pad_conn(conn_out)
    mem_in_f = mem_in.reshape(-1)
    mem_state_f = mem_state.reshape(-1)
    mem_out_f = mem_out.reshape(-1)

    layer_in = _make_layer("input", NBITS, INPUT_BITS)
    layer_state = _make_layer("state", NBITS, NBITS)
    layer_out = _make_layer("out", N_OUT, NBITS)

    bits = layer_in(xT, conn_in_p, mem_in_f)
    for _ in range(4):
        bits = layer_state(bits, conn_state_p, mem_state_f)
    out_t = layer_out(bits, conn_out_p, mem_out_f)
    return out_t.T


# speedup vs baseline: 1.4338x; 1.4338x over previous
import jax
import jax.numpy as jnp
from jax import lax
from jax.experimental import pallas as pl
from jax.experimental.pallas import tpu as pltpu
from jax.experimental.pallas import tpu_sc as plsc

NC, NS, LANES = 2, 16, 16
NW = NC * NS

INPUT_BITS = 4096
N_IN = 1024
N_STATE = 512
N_OUT = 512
NBITS = N_IN + N_STATE
K = 14
KP = 16
MEM = 1 << K
B = 1024
NCHUNK = B // LANES

_MESH = plsc.VectorSubcoreMesh(
    core_axis_name="c", subcore_axis_name="s", num_cores=NC, num_subcores=NS
)


def _make_layer(mode):
    R = 32 if mode == "input" else 16
    out_rows = N_OUT if mode == "out" else NBITS
    out_dtype = jnp.float32 if mode == "out" else jnp.int32

    def body(src, conn, mem, out, conn_l, rows_a, rows_b, tab_a, tab_b,
             orow_a, orow_b, zbuf, sem_a, sem_b, osem_a, osem_b):
        wid = lax.axis_index("s") * NC + lax.axis_index("c")
        rows_ = (rows_a, rows_b)
        tab_ = (tab_a, tab_b)
        orow_ = (orow_a, orow_b)
        sem_ = (sem_a, sem_b)
        osem_ = (osem_a, osem_b)

        pltpu.sync_copy(conn.at[pl.ds(wid * R, R)], conn_l)

        def start(j, s):
            neuron = j * NW + wid
            pltpu.async_copy(src.at[conn_l.at[j]], rows_[s], sem_[s])
            pltpu.async_copy(mem.at[neuron], tab_[s], sem_[s])

        def wait_slot(s):
            pltpu.make_async_copy(src.at[pl.ds(0, KP)], rows_[s], sem_[s]).wait()
            pltpu.make_async_copy(mem.at[0], tab_[s], sem_[s]).wait()

        def compute(j, s):
            rows16, tab, orow = rows_[s], tab_[s], orow_[s]
            neuron = j * NW + wid
            out_row = neuron + (N_IN if mode == "state" else 0)
            for c in range(NCHUNK):
                a = rows16[0, pl.ds(c * LANES, LANES)]
                for k in range(1, K):
                    a = a + (rows16[k, pl.ds(c * LANES, LANES)] << k)
                v = plsc.load_gather(tab, [a])
                if mode == "out":
                    orow[pl.ds(c * LANES, LANES)] = v
                else:
                    orow[pl.ds(c * LANES, LANES)] = jnp.where(
                        v > 0.5, jnp.int32(1), jnp.int32(0))
            pltpu.async_copy(orow, out.at[out_row], osem_[s])

        def drain_out(s):
            pltpu.make_async_copy(orow_[s], out.at[0], osem_[s]).wait()

        if mode == "input":
            for r in range(16):
                for c in range(NCHUNK):
                    zbuf[r, pl.ds(c * LANES, LANES)] = jnp.zeros(
                        (LANES,), jnp.int32)
            pltpu.sync_copy(zbuf, out.at[pl.ds(N_IN + wid * 16, 16)])
        elif mode == "state":
            for p in range(2):
                sl = pl.ds(wid * 32 + p * 16, 16)
                pltpu.sync_copy(src.at[sl], zbuf)
                pltpu.sync_copy(zbuf, out.at[sl])

        start(0, 0)

        @pl.loop(0, R, step=2)
        def _(j):
            start(j + 1, 1)
            wait_slot(0)

            @pl.when(j >= 2)
            def _():
                drain_out(0)

            compute(j, 0)

            @pl.when(j + 2 < R)
            def _():
                start(j + 2, 0)

            wait_slot(1)

            @pl.when(j >= 2)
            def _():
                drain_out(1)

            compute(j + 1, 1)

        drain_out(0)
        drain_out(1)

    return pl.kernel(
        body,
        out_type=jax.ShapeDtypeStruct((out_rows, B), out_dtype),
        mesh=_MESH,
        compiler_params=pltpu.CompilerParams(needs_layout_passes=False),
        scratch_types=[
            pltpu.VMEM((R, KP), jnp.int32),
            pltpu.VMEM((KP, B), jnp.int32),
            pltpu.VMEM((KP, B), jnp.int32),
            pltpu.VMEM((MEM,), jnp.float32),
            pltpu.VMEM((MEM,), jnp.float32),
            pltpu.VMEM((B,), out_dtype),
            pltpu.VMEM((B,), out_dtype),
            pltpu.VMEM((16, B), jnp.int32),
            pltpu.SemaphoreType.DMA,
            pltpu.SemaphoreType.DMA,
            pltpu.SemaphoreType.DMA,
            pltpu.SemaphoreType.DMA,
        ],
        name=f"ram_layer_{mode}",
    )


def _pad_conn(conn, r):
    n, k = conn.shape
    p = jnp.concatenate(
        [conn.astype(jnp.int32), jnp.zeros((n, KP - k), jnp.int32)], axis=1)
    return p.reshape(r, NW, KP).swapaxes(0, 1).reshape(n, KP)


@jax.jit
def kernel(x, conn_in, conn_state, conn_out, mem_in, mem_state, mem_out):
    xT = x.astype(jnp.int32).T
    conn_in_p = _pad_conn(conn_in, 32)
    conn_state_p = _pad_conn(conn_state, 16)
    conn_out_p = _pad_conn(conn_out, 16)

    layer_in = _make_layer("input")
    layer_state = _make_layer("state")
    layer_out = _make_layer("out")

    bits = layer_in(xT, conn_in_p, mem_in)
    for _ in range(4):
        bits = layer_state(bits, conn_state_p, mem_state)
    out_t = layer_out(bits, conn_out_p, mem_out)
    return out_t.T
